# baseline (speedup 1.0000x reference)
import functools

import jax
import jax.numpy as jnp
from jax.experimental import pallas as pl
from jax.experimental.pallas import tpu as pltpu

N = 4096
D = 128
V = 1000
NBLK = 16
BLK = N // NBLK


def _gcn_kernel(a_hbm, x_ref, w1_ref, b1_ref, w2_ref, b2_ref, wd_ref, bd_ref,
                out_hbm, a_bf, z_ref, h1_ref, vin, out_buf, sem_in, sem_out):
    p = pl.program_id(0)
    i = pl.program_id(1)
    slot = jax.lax.rem(i, 2)
    nslot = jax.lax.rem(i + 1, 2)

    @pl.when(p == 0)
    def _phase0():
        @pl.when(i == 0)
        def _first():
            pltpu.make_async_copy(a_hbm.at[pl.ds(0, BLK), :], vin.at[0],
                                  sem_in.at[0]).start()
            z1 = jnp.dot(x_ref[...], w1_ref[...],
                         preferred_element_type=jnp.float32)
            z_ref[...] = z1.astype(jnp.bfloat16)

        @pl.when(i + 1 < NBLK)
        def _prefetch():
            pltpu.make_async_copy(a_hbm.at[pl.ds((i + 1) * BLK, BLK), :],
                                  vin.at[nslot], sem_in.at[nslot]).start()

        pltpu.make_async_copy(a_hbm.at[pl.ds(i * BLK, BLK), :], vin.at[slot],
                              sem_in.at[slot]).wait()
        ab = vin[slot].astype(jnp.bfloat16)
        a_bf[pl.ds(i * BLK, BLK), :] = ab
        h = jnp.dot(ab, z_ref[...], preferred_element_type=jnp.float32)
        h = jnp.maximum(h + b1_ref[...], 0.0)
        h1_ref[pl.ds(i * BLK, BLK), :] = h.astype(jnp.bfloat16)

    @pl.when(p == 1)
    def _phase1():
        @pl.when(i == 0)
        def _init_z2():
            z2 = jnp.dot(h1_ref[...], w2_ref[...].astype(jnp.bfloat16),
                         preferred_element_type=jnp.float32)
            z_ref[...] = z2.astype(jnp.bfloat16)

        h2 = jnp.dot(a_bf[pl.ds(i * BLK, BLK), :], z_ref[...],
                     preferred_element_type=jnp.float32)
        h2 = jnp.maximum(h2 + b2_ref[...], 0.0)
        out = jnp.dot(h2.astype(jnp.bfloat16), wd_ref[...].astype(jnp.bfloat16),
                      preferred_element_type=jnp.float32)

        @pl.when(i >= 2)
        def _wait_prev():
            pltpu.make_async_copy(out_buf.at[slot],
                                  out_hbm.at[pl.ds((i - 2) * BLK, BLK), :],
                                  sem_out.at[slot]).wait()

        out_buf[slot] = out + bd_ref[...]
        pltpu.make_async_copy(out_buf.at[slot],
                              out_hbm.at[pl.ds(i * BLK, BLK), :],
                              sem_out.at[slot]).start()

        @pl.when(i == NBLK - 1)
        def _drain():
            pltpu.make_async_copy(out_buf.at[nslot],
                                  out_hbm.at[pl.ds((i - 1) * BLK, BLK), :],
                                  sem_out.at[nslot]).wait()
            pltpu.make_async_copy(out_buf.at[slot],
                                  out_hbm.at[pl.ds(i * BLK, BLK), :],
                                  sem_out.at[slot]).wait()


@functools.partial(jax.jit, static_argnames=())
def kernel(feature, graph, W1, b1, W2, b2, Wd, bd):
    b1r = b1.reshape(1, D)
    b2r = b2.reshape(1, D)
    bdr = bd.reshape(1, V)

    grid = (2, NBLK)
    out = pl.pallas_call(
        _gcn_kernel,
        grid=grid,
        in_specs=[
            pl.BlockSpec(memory_space=pl.ANY),
            pl.BlockSpec((N, D), lambda p, i: (0, 0)),
            pl.BlockSpec((D, D), lambda p, i: (0, 0)),
            pl.BlockSpec((1, D), lambda p, i: (0, 0)),
            pl.BlockSpec((D, D), lambda p, i: (0, 0)),
            pl.BlockSpec((1, D), lambda p, i: (0, 0)),
            pl.BlockSpec((D, V), lambda p, i: (0, 0)),
            pl.BlockSpec((1, V), lambda p, i: (0, 0)),
        ],
        out_specs=pl.BlockSpec(memory_space=pl.ANY),
        out_shape=jax.ShapeDtypeStruct((N, V), jnp.float32),
        scratch_shapes=[
            pltpu.VMEM((N, N), jnp.bfloat16),
            pltpu.VMEM((N, D), jnp.bfloat16),
            pltpu.VMEM((N, D), jnp.bfloat16),
            pltpu.VMEM((2, BLK, N), jnp.float32),
            pltpu.VMEM((2, BLK, V), jnp.float32),
            pltpu.SemaphoreType.DMA((2,)),
            pltpu.SemaphoreType.DMA((2,)),
        ],
        compiler_params=pltpu.CompilerParams(
            dimension_semantics=("arbitrary", "arbitrary"),
            vmem_limit_bytes=110 * 1024 * 1024,
        ),
    )(graph, feature, W1, b1r, W2, b2r, Wd, bdr)
    return out


# speedup vs baseline: 1.0631x; 1.0631x over previous
import functools

import jax
import jax.numpy as jnp
from jax.experimental import pallas as pl
from jax.experimental.pallas import tpu as pltpu

N = 4096
D = 128
V = 1000
NBLK = 16
BLK = N // NBLK
SLOTS = 4


def _gcn_kernel(a_hbm, x_ref, w1_ref, b1_ref, w2_ref, b2_ref, wd_ref, bd_ref,
                out_hbm, a_bf, z_ref, h1_ref, vin, out_buf, sem_in, sem_out):
    p = pl.program_id(0)
    i = pl.program_id(1)
    slot = jax.lax.rem(i, SLOTS)
    oslot = jax.lax.rem(i, 2)
    onslot = jax.lax.rem(i + 1, 2)

    @pl.when(p == 0)
    def _phase0():
        @pl.when(i == 0)
        def _first():
            for s in range(SLOTS):
                pltpu.make_async_copy(a_hbm.at[pl.ds(s * BLK, BLK), :],
                                      vin.at[s], sem_in.at[s]).start()
            z1 = jnp.dot(x_ref[...], w1_ref[...],
                         preferred_element_type=jnp.float32)
            z_ref[...] = z1.astype(jnp.bfloat16)

        @pl.when(jnp.logical_and(i > 0, i + SLOTS - 1 < NBLK))
        def _prefetch():
            pf = i + SLOTS - 1
            pltpu.make_async_copy(a_hbm.at[pl.ds(pf * BLK, BLK), :],
                                  vin.at[jax.lax.rem(pf, SLOTS)],
                                  sem_in.at[jax.lax.rem(pf, SLOTS)]).start()

        pltpu.make_async_copy(a_hbm.at[pl.ds(i * BLK, BLK), :], vin.at[slot],
                              sem_in.at[slot]).wait()
        ab = vin[slot].astype(jnp.bfloat16)
        a_bf[pl.ds(i * BLK, BLK), :] = ab
        h = jnp.dot(ab, z_ref[...], preferred_element_type=jnp.float32)
        h = jnp.maximum(h + b1_ref[...], 0.0)
        h1_ref[pl.ds(i * BLK, BLK), :] = h.astype(jnp.bfloat16)

    @pl.when(p == 1)
    def _phase1():
        @pl.when(i == 0)
        def _init_z2():
            z2 = jnp.dot(h1_ref[...], w2_ref[...].astype(jnp.bfloat16),
                         preferred_element_type=jnp.float32)
            z_ref[...] = z2.astype(jnp.bfloat16)

        h2 = jnp.dot(a_bf[pl.ds(i * BLK, BLK), :], z_ref[...],
                     preferred_element_type=jnp.float32)
        h2 = jnp.maximum(h2 + b2_ref[...], 0.0)
        out = jnp.dot(h2.astype(jnp.bfloat16), wd_ref[...].astype(jnp.bfloat16),
                      preferred_element_type=jnp.float32)

        @pl.when(i >= 2)
        def _wait_prev():
            pltpu.make_async_copy(out_buf.at[oslot],
                                  out_hbm.at[pl.ds((i - 2) * BLK, BLK), :],
                                  sem_out.at[oslot]).wait()

        out_buf[oslot] = out + bd_ref[...]
        pltpu.make_async_copy(out_buf.at[oslot],
                              out_hbm.at[pl.ds(i * BLK, BLK), :],
                              sem_out.at[oslot]).start()

        @pl.when(i == NBLK - 1)
        def _drain():
            pltpu.make_async_copy(out_buf.at[onslot],
                                  out_hbm.at[pl.ds((i - 1) * BLK, BLK), :],
                                  sem_out.at[onslot]).wait()
            pltpu.make_async_copy(out_buf.at[oslot],
                                  out_hbm.at[pl.ds(i * BLK, BLK), :],
                                  sem_out.at[oslot]).wait()


@functools.partial(jax.jit, static_argnames=())
def kernel(feature, graph, W1, b1, W2, b2, Wd, bd):
    b1r = b1.reshape(1, D)
    b2r = b2.reshape(1, D)
    bdr = bd.reshape(1, V)

    grid = (2, NBLK)
    out = pl.pallas_call(
        _gcn_kernel,
        grid=grid,
        in_specs=[
            pl.BlockSpec(memory_space=pl.ANY),
            pl.BlockSpec((N, D), lambda p, i: (0, 0)),
            pl.BlockSpec((D, D), lambda p, i: (0, 0)),
            pl.BlockSpec((1, D), lambda p, i: (0, 0)),
            pl.BlockSpec((D, D), lambda p, i: (0, 0)),
            pl.BlockSpec((1, D), lambda p, i: (0, 0)),
            pl.BlockSpec((D, V), lambda p, i: (0, 0)),
            pl.BlockSpec((1, V), lambda p, i: (0, 0)),
        ],
        out_specs=pl.BlockSpec(memory_space=pl.ANY),
        out_shape=jax.ShapeDtypeStruct((N, V), jnp.float32),
        scratch_shapes=[
            pltpu.VMEM((N, N), jnp.bfloat16),
            pltpu.VMEM((N, D), jnp.bfloat16),
            pltpu.VMEM((N, D), jnp.bfloat16),
            pltpu.VMEM((SLOTS, BLK, N), jnp.float32),
            pltpu.VMEM((2, BLK, V), jnp.float32),
            pltpu.SemaphoreType.DMA((SLOTS,)),
            pltpu.SemaphoreType.DMA((2,)),
        ],
        compiler_params=pltpu.CompilerParams(
            dimension_semantics=("arbitrary", "arbitrary"),
            vmem_limit_bytes=110 * 1024 * 1024,
        ),
    )(graph, feature, W1, b1r, W2, b2r, Wd, bdr)
    return out
